# Initial kernel scaffold; baseline (speedup 1.0000x reference)
import jax
import jax.numpy as jnp
from jax.experimental import pallas as pl


def kernel(text, embedding_table, W1, b1, W2, b2):
    raise NotImplementedError("write your pallas kernel here")


# speedup vs baseline: 23.6333x; 23.6333x over previous
import functools

import jax
import jax.numpy as jnp
from jax import lax
from jax.experimental import pallas as pl
from jax.experimental.pallas import tpu as pltpu
from jax.experimental.pallas import tpu_sc as plsc

_VOCAB = 1000000
_EMBED = 32
_HIDDEN = 128
_BATCH = 4096
_SEQ = 200
_NTOK = _BATCH * _SEQ

_BLK = 8000
_NBLK = _VOCAB // _BLK

_GLANES = 128


def _mlp_body(t_ref, w1_ref, b1_ref, w2_ref, b2_ref, y_ref):
    t = t_ref[...]
    h = jnp.dot(t, w1_ref[...], preferred_element_type=jnp.float32)
    h = jnp.maximum(h + b1_ref[...], 0.0)
    y = jnp.sum(h * w2_ref[...], axis=1) + b2_ref[0, 0]
    y_ref[...] = y.reshape(1, 1, _BLK)


def _precompute_y(table, W1, b1, W2, b2):
    y = pl.pallas_call(
        _mlp_body,
        grid=(_NBLK,),
        in_specs=[
            pl.BlockSpec((_BLK, _EMBED), lambda i: (i, 0)),
            pl.BlockSpec((_EMBED, _HIDDEN), lambda i: (0, 0)),
            pl.BlockSpec((1, _HIDDEN), lambda i: (0, 0)),
            pl.BlockSpec((1, _HIDDEN), lambda i: (0, 0)),
            pl.BlockSpec((1, 1), lambda i: (0, 0)),
        ],
        out_specs=pl.BlockSpec((1, 1, _BLK), lambda i: (i, 0, 0)),
        out_shape=jax.ShapeDtypeStruct((_NBLK, 1, _BLK), jnp.float32),
    )(table, W1, b1.reshape(1, _HIDDEN), W2.reshape(1, _HIDDEN),
      b2.reshape(1, 1))
    return y.reshape(_VOCAB)


@functools.lru_cache(maxsize=None)
def _build_gather():
    info = plsc.get_sparse_core_info()
    nc, ns = info.num_cores, info.num_subcores
    nw = nc * ns
    chunks = _NTOK // (nw * _GLANES)
    mesh = plsc.VectorSubcoreMesh(core_axis_name="c", subcore_axis_name="s")

    @functools.partial(
        pl.kernel,
        mesh=mesh,
        out_type=jax.ShapeDtypeStruct((nw, chunks, _GLANES), jnp.float32),
        scratch_types=[
            pltpu.VMEM((chunks, _GLANES), jnp.int32),
            pltpu.VMEM((chunks, _GLANES), jnp.float32),
            pltpu.SemaphoreType.DMA,
        ],
    )
    def gather_k(idx_hbm, tab_hbm, out_hbm, idx_v, vals_v, sem):
        wid = lax.axis_index("s") * nc + lax.axis_index("c")
        pltpu.sync_copy(idx_hbm.at[wid], idx_v)

        def fire(j, c):
            pltpu.make_async_copy(
                tab_hbm.at[idx_v.at[j]], vals_v.at[j], sem).start()
            return c

        lax.fori_loop(0, chunks, fire, 0)

        def drain(j, c):
            pltpu.make_async_copy(
                tab_hbm.at[idx_v.at[j]], vals_v.at[j], sem).wait()
            return c

        lax.fori_loop(0, chunks, drain, 0)
        pltpu.sync_copy(vals_v, out_hbm.at[wid])

    return gather_k, nw, chunks


def kernel(text, embedding_table, W1, b1, W2, b2):
    y = _precompute_y(embedding_table, W1, b1, W2, b2)
    gather_k, nw, chunks = _build_gather()
    idx = text.reshape(nw, chunks, _GLANES)
    out = gather_k(idx, y)
    return out.reshape(_BATCH, _SEQ, 1)


# speedup vs baseline: 24.7543x; 1.0474x over previous
import functools

import jax
import jax.numpy as jnp
from jax import lax
from jax.experimental import pallas as pl
from jax.experimental.pallas import tpu as pltpu
from jax.experimental.pallas import tpu_sc as plsc

_VOCAB = 1000000
_EMBED = 32
_HIDDEN = 128
_BATCH = 4096
_SEQ = 200
_NTOK = _BATCH * _SEQ

_BLK = 8000
_NBLK = _VOCAB // _BLK

_GLANES = 128


def _mlp_body(t_ref, w1_ref, b1_ref, w2_ref, b2_ref, y_ref):
    t = t_ref[...]
    h = jnp.dot(t, w1_ref[...], preferred_element_type=jnp.float32)
    h = jnp.maximum(h + b1_ref[...], 0.0)
    y_ref[...] = (jnp.dot(h, w2_ref[...],
                          preferred_element_type=jnp.float32)
                  + b2_ref[0, 0])


def _precompute_y(table, W1, b1, W2, b2):
    y = pl.pallas_call(
        _mlp_body,
        grid=(_NBLK,),
        in_specs=[
            pl.BlockSpec((_BLK, _EMBED), lambda i: (i, 0)),
            pl.BlockSpec((_EMBED, _HIDDEN), lambda i: (0, 0)),
            pl.BlockSpec((1, _HIDDEN), lambda i: (0, 0)),
            pl.BlockSpec((_HIDDEN, 1), lambda i: (0, 0)),
            pl.BlockSpec((1, 1), lambda i: (0, 0)),
        ],
        out_specs=pl.BlockSpec((_BLK, 1), lambda i: (i, 0)),
        out_shape=jax.ShapeDtypeStruct((_VOCAB, 1), jnp.float32),
    )(table, W1, b1.reshape(1, _HIDDEN), W2, b2.reshape(1, 1))
    return y.reshape(_VOCAB)


@functools.lru_cache(maxsize=None)
def _build_gather():
    info = plsc.get_sparse_core_info()
    nc, ns = info.num_cores, info.num_subcores
    nw = nc * ns
    chunks = _NTOK // (nw * _GLANES)
    mesh = plsc.VectorSubcoreMesh(core_axis_name="c", subcore_axis_name="s")

    @functools.partial(
        pl.kernel,
        mesh=mesh,
        out_type=jax.ShapeDtypeStruct((nw, chunks, _GLANES), jnp.float32),
        scratch_types=[
            pltpu.VMEM((chunks, _GLANES), jnp.int32),
            pltpu.VMEM((chunks, _GLANES), jnp.float32),
            pltpu.SemaphoreType.DMA,
        ],
    )
    def gather_k(idx_hbm, tab_hbm, out_hbm, idx_v, vals_v, sem):
        wid = lax.axis_index("s") * nc + lax.axis_index("c")
        pltpu.sync_copy(idx_hbm.at[wid], idx_v)

        def fire(j, c):
            pltpu.make_async_copy(
                tab_hbm.at[idx_v.at[j]], vals_v.at[j], sem).start()
            return c

        lax.fori_loop(0, chunks, fire, 0)

        def drain(j, c):
            pltpu.make_async_copy(
                tab_hbm.at[idx_v.at[j]], vals_v.at[j], sem).wait()
            return c

        lax.fori_loop(0, chunks, drain, 0)
        pltpu.sync_copy(vals_v, out_hbm.at[wid])

    return gather_k, nw, chunks


def kernel(text, embedding_table, W1, b1, W2, b2):
    y = _precompute_y(embedding_table, W1, b1, W2, b2)
    gather_k, nw, chunks = _build_gather()
    idx = text.reshape(nw, chunks, _GLANES)
    out = gather_k(idx, y)
    return out.reshape(_BATCH, _SEQ, 1)


# speedup vs baseline: 96.2947x; 3.8900x over previous
import functools

import jax
import jax.numpy as jnp
from jax import lax
from jax.experimental import pallas as pl
from jax.experimental.pallas import tpu as pltpu
from jax.experimental.pallas import tpu_sc as plsc

_VOCAB = 1000000
_EMBED = 32
_HIDDEN = 128
_BATCH = 4096
_SEQ = 200
_NTOK = _BATCH * _SEQ

_BLK = 8192
_NBLK = (_VOCAB + _BLK - 1) // _BLK

_GLANES = 128


def _mlp_body(tt_ref, w1t_ref, b1_ref, w2t_ref, b2_ref, y_ref):
    tt = tt_ref[...]
    h = jnp.dot(w1t_ref[...], tt, preferred_element_type=jnp.float32)
    h = jnp.maximum(h + b1_ref[...], 0.0)
    y = (jnp.dot(w2t_ref[...], h, preferred_element_type=jnp.float32)
         + b2_ref[0, 0])
    y_ref[...] = y.reshape(_BLK)


def _precompute_y(table, W1, b1, W2, b2):
    y = pl.pallas_call(
        _mlp_body,
        grid=(_NBLK,),
        in_specs=[
            pl.BlockSpec((_EMBED, _BLK), lambda i: (0, i)),
            pl.BlockSpec((_HIDDEN, _EMBED), lambda i: (0, 0)),
            pl.BlockSpec((_HIDDEN, 1), lambda i: (0, 0)),
            pl.BlockSpec((1, _HIDDEN), lambda i: (0, 0)),
            pl.BlockSpec((1, 1), lambda i: (0, 0)),
        ],
        out_specs=pl.BlockSpec((_BLK,), lambda i: (i,)),
        out_shape=jax.ShapeDtypeStruct((_VOCAB,), jnp.float32),
    )(table.T, W1.T, b1.reshape(_HIDDEN, 1), W2.T, b2.reshape(1, 1))
    return y


@functools.lru_cache(maxsize=None)
def _build_gather():
    info = plsc.get_sparse_core_info()
    nc, ns = info.num_cores, info.num_subcores
    nw = nc * ns
    chunks = _NTOK // (nw * _GLANES)
    mesh = plsc.VectorSubcoreMesh(core_axis_name="c", subcore_axis_name="s")

    @functools.partial(
        pl.kernel,
        mesh=mesh,
        out_type=jax.ShapeDtypeStruct((nw, chunks, _GLANES), jnp.float32),
        scratch_types=[
            pltpu.VMEM((chunks, _GLANES), jnp.int32),
            pltpu.VMEM((chunks, _GLANES), jnp.float32),
            pltpu.SemaphoreType.DMA,
        ],
    )
    def gather_k(idx_hbm, tab_hbm, out_hbm, idx_v, vals_v, sem):
        wid = lax.axis_index("s") * nc + lax.axis_index("c")
        pltpu.sync_copy(idx_hbm.at[wid], idx_v)

        def fire(j, c):
            pltpu.make_async_copy(
                tab_hbm.at[idx_v.at[j]], vals_v.at[j], sem).start()
            return c

        lax.fori_loop(0, chunks, fire, 0)

        def drain(j, c):
            pltpu.make_async_copy(
                tab_hbm.at[idx_v.at[j]], vals_v.at[j], sem).wait()
            return c

        lax.fori_loop(0, chunks, drain, 0)
        pltpu.sync_copy(vals_v, out_hbm.at[wid])

    return gather_k, nw, chunks


def kernel(text, embedding_table, W1, b1, W2, b2):
    y = _precompute_y(embedding_table, W1, b1, W2, b2)
    gather_k, nw, chunks = _build_gather()
    idx = text.reshape(nw, chunks, _GLANES)
    out = gather_k(idx, y)
    return out.reshape(_BATCH, _SEQ, 1)


# speedup vs baseline: 100.6464x; 1.0452x over previous
import functools

import jax
import jax.numpy as jnp
from jax import lax
from jax.experimental import pallas as pl
from jax.experimental.pallas import tpu as pltpu
from jax.experimental.pallas import tpu_sc as plsc

_VOCAB = 1000000
_EMBED = 32
_HIDDEN = 128
_BATCH = 4096
_SEQ = 200
_NTOK = _BATCH * _SEQ

_BLK = 8192
_NBLK = (_VOCAB + _BLK - 1) // _BLK

_GLANES = 128


def _mlp_body(tt_ref, w1t_ref, b1_ref, w2t_ref, b2_ref, y_ref):
    tt = tt_ref[...]
    h = jnp.dot(w1t_ref[...], tt, preferred_element_type=jnp.float32)
    h = jnp.maximum(h + b1_ref[...], 0.0)
    y = (jnp.dot(w2t_ref[...], h, preferred_element_type=jnp.float32)
         + b2_ref[0, 0])
    y_ref[...] = y.reshape(_BLK)


def _precompute_y(table, W1, b1, W2, b2):
    y = pl.pallas_call(
        _mlp_body,
        grid=(_NBLK,),
        in_specs=[
            pl.BlockSpec((_EMBED, _BLK), lambda i: (0, i)),
            pl.BlockSpec((_HIDDEN, _EMBED), lambda i: (0, 0)),
            pl.BlockSpec((_HIDDEN, 1), lambda i: (0, 0)),
            pl.BlockSpec((1, _HIDDEN), lambda i: (0, 0)),
            pl.BlockSpec((1, 1), lambda i: (0, 0)),
        ],
        out_specs=pl.BlockSpec((_BLK,), lambda i: (i,)),
        out_shape=jax.ShapeDtypeStruct((_VOCAB,), jnp.float32),
    )(table.T, W1.T, b1.reshape(_HIDDEN, 1), W2.T, b2.reshape(1, 1))
    return y


@functools.lru_cache(maxsize=None)
def _build_gather():
    info = plsc.get_sparse_core_info()
    nc, ns = info.num_cores, info.num_subcores
    nw = nc * ns
    chunks = _NTOK // (nw * _GLANES)
    mesh = plsc.VectorSubcoreMesh(core_axis_name="c", subcore_axis_name="s")

    @functools.partial(
        pl.kernel,
        mesh=mesh,
        out_type=jax.ShapeDtypeStruct((nw, chunks, _GLANES), jnp.float32),
        scratch_types=[
            pltpu.VMEM((chunks, _GLANES), jnp.int32),
            pltpu.VMEM((chunks, _GLANES), jnp.float32),
            pltpu.SemaphoreType.DMA,
        ],
    )
    def gather_k(idx_hbm, tab_hbm, out_hbm, idx_v, vals_v, sem):
        wid = lax.axis_index("s") * nc + lax.axis_index("c")
        pltpu.sync_copy(idx_hbm.at[wid], idx_v)

        def fire(j, c):
            pltpu.make_async_copy(
                tab_hbm.at[idx_v.at[j]], vals_v.at[j], sem).start()
            return c

        lax.fori_loop(0, chunks, fire, 0)

        def drain(j, c):
            pltpu.make_async_copy(
                tab_hbm.at[idx_v.at[j]], vals_v.at[j], sem).wait()
            return c

        lax.fori_loop(0, chunks, drain, 0)
        pltpu.sync_copy(vals_v, out_hbm.at[wid])

    return gather_k, nw, chunks


def kernel(text, embedding_table, W1, b1, W2, b2):
    y = _precompute_y(embedding_table, W1, b1, W2, b2)
    gather_k, nw, chunks = _build_gather()
    idx = text.T.reshape(nw, chunks, _GLANES)
    out = gather_k(idx, y)
    return out.reshape(_SEQ, _BATCH).T.reshape(_BATCH, _SEQ, 1)


# speedup vs baseline: 115.5658x; 1.1482x over previous
import functools

import jax
import jax.numpy as jnp
from jax import lax
from jax.experimental import pallas as pl
from jax.experimental.pallas import tpu as pltpu
from jax.experimental.pallas import tpu_sc as plsc

_VOCAB = 1000000
_EMBED = 32
_HIDDEN = 128
_BATCH = 4096
_SEQ = 200
_NTOK = _BATCH * _SEQ

_BLK = 16384
_NBLK = (_VOCAB + _BLK - 1) // _BLK

_GLANES = 128


def _mlp_body(tt_ref, w1t_ref, b1_ref, w2t_ref, b2_ref, y_ref):
    tt = tt_ref[...]
    h = jnp.dot(w1t_ref[...], tt, preferred_element_type=jnp.float32)
    h = jnp.maximum(h + b1_ref[...], 0.0)
    y = (jnp.dot(w2t_ref[...], h, preferred_element_type=jnp.float32)
         + b2_ref[0, 0])
    y_ref[...] = y.reshape(_BLK)


def _precompute_y(table, W1, b1, W2, b2):
    y = pl.pallas_call(
        _mlp_body,
        grid=(_NBLK,),
        in_specs=[
            pl.BlockSpec((_EMBED, _BLK), lambda i: (0, i)),
            pl.BlockSpec((_HIDDEN, _EMBED), lambda i: (0, 0)),
            pl.BlockSpec((_HIDDEN, 1), lambda i: (0, 0)),
            pl.BlockSpec((1, _HIDDEN), lambda i: (0, 0)),
            pl.BlockSpec((1, 1), lambda i: (0, 0)),
        ],
        out_specs=pl.BlockSpec((_BLK,), lambda i: (i,)),
        out_shape=jax.ShapeDtypeStruct((_VOCAB,), jnp.float32),
    )(table.T, W1.T, b1.reshape(_HIDDEN, 1), W2.T, b2.reshape(1, 1))
    return y


@functools.lru_cache(maxsize=None)
def _build_gather():
    info = plsc.get_sparse_core_info()
    nc, ns = info.num_cores, info.num_subcores
    nw = nc * ns
    chunks = _NTOK // (nw * _GLANES)
    mesh = plsc.VectorSubcoreMesh(core_axis_name="c", subcore_axis_name="s")

    @functools.partial(
        pl.kernel,
        mesh=mesh,
        out_type=jax.ShapeDtypeStruct((nw, chunks, _GLANES), jnp.float32),
        scratch_types=[
            pltpu.VMEM((chunks, _GLANES), jnp.int32),
            pltpu.VMEM((chunks, _GLANES), jnp.float32),
            pltpu.SemaphoreType.DMA,
        ],
    )
    def gather_k(idx_hbm, tab_hbm, out_hbm, idx_v, vals_v, sem):
        wid = lax.axis_index("s") * nc + lax.axis_index("c")
        pltpu.sync_copy(idx_hbm.at[wid], idx_v)

        def fire(j, c):
            pltpu.make_async_copy(
                tab_hbm.at[idx_v.at[j]], vals_v.at[j], sem).start()
            return c

        lax.fori_loop(0, chunks, fire, 0)

        def drain(j, c):
            pltpu.make_async_copy(
                tab_hbm.at[idx_v.at[j]], vals_v.at[j], sem).wait()
            return c

        lax.fori_loop(0, chunks, drain, 0)
        pltpu.sync_copy(vals_v, out_hbm.at[wid])

    return gather_k, nw, chunks


def kernel(text, embedding_table, W1, b1, W2, b2):
    y = _precompute_y(embedding_table, W1, b1, W2, b2)
    gather_k, nw, chunks = _build_gather()
    idx = text.T.reshape(nw, chunks, _GLANES)
    out = gather_k(idx, y)
    return out.reshape(_SEQ, _BATCH).T.reshape(_BATCH, _SEQ, 1)


# speedup vs baseline: 125.7850x; 1.0884x over previous
import functools

import jax
import jax.numpy as jnp
from jax import lax
from jax.experimental import pallas as pl
from jax.experimental.pallas import tpu as pltpu
from jax.experimental.pallas import tpu_sc as plsc

_VOCAB = 1000000
_EMBED = 32
_HIDDEN = 128
_BATCH = 4096
_SEQ = 200
_NTOK = _BATCH * _SEQ

_BLK = 16384
_NBLK = (_VOCAB + _BLK - 1) // _BLK

_GLANES = 128


def _mlp_body(tt_ref, w1t_ref, b1_ref, w2c_ref, b2_ref, y_ref):
    tt = tt_ref[...]
    h = jnp.dot(w1t_ref[...], tt, preferred_element_type=jnp.float32)
    h = jnp.maximum(h + b1_ref[...], 0.0)
    y = jnp.sum(h * w2c_ref[...], axis=0) + b2_ref[0, 0]
    y_ref[...] = y


def _precompute_y(table, W1, b1, W2, b2):
    y = pl.pallas_call(
        _mlp_body,
        grid=(_NBLK,),
        in_specs=[
            pl.BlockSpec((_EMBED, _BLK), lambda i: (0, i)),
            pl.BlockSpec((_HIDDEN, _EMBED), lambda i: (0, 0)),
            pl.BlockSpec((_HIDDEN, 1), lambda i: (0, 0)),
            pl.BlockSpec((_HIDDEN, 1), lambda i: (0, 0)),
            pl.BlockSpec((1, 1), lambda i: (0, 0)),
        ],
        out_specs=pl.BlockSpec((_BLK,), lambda i: (i,)),
        out_shape=jax.ShapeDtypeStruct((_VOCAB,), jnp.float32),
    )(table.T, W1.T, b1.reshape(_HIDDEN, 1), W2, b2.reshape(1, 1))
    return y


@functools.lru_cache(maxsize=None)
def _build_gather():
    info = plsc.get_sparse_core_info()
    nc, ns = info.num_cores, info.num_subcores
    nw = nc * ns
    chunks = _NTOK // (nw * _GLANES)
    mesh = plsc.VectorSubcoreMesh(core_axis_name="c", subcore_axis_name="s")

    @functools.partial(
        pl.kernel,
        mesh=mesh,
        out_type=jax.ShapeDtypeStruct((nw, chunks, _GLANES), jnp.float32),
        scratch_types=[
            pltpu.VMEM((chunks, _GLANES), jnp.int32),
            pltpu.VMEM((chunks, _GLANES), jnp.float32),
            pltpu.SemaphoreType.DMA,
        ],
    )
    def gather_k(idx_hbm, tab_hbm, out_hbm, idx_v, vals_v, sem):
        wid = lax.axis_index("s") * nc + lax.axis_index("c")
        pltpu.sync_copy(idx_hbm.at[wid], idx_v)

        def fire(j, c):
            pltpu.make_async_copy(
                tab_hbm.at[idx_v.at[j]], vals_v.at[j], sem).start()
            return c

        lax.fori_loop(0, chunks, fire, 0)

        def drain(j, c):
            pltpu.make_async_copy(
                tab_hbm.at[idx_v.at[j]], vals_v.at[j], sem).wait()
            return c

        lax.fori_loop(0, chunks, drain, 0)
        pltpu.sync_copy(vals_v, out_hbm.at[wid])

    return gather_k, nw, chunks


def kernel(text, embedding_table, W1, b1, W2, b2):
    y = _precompute_y(embedding_table, W1, b1, W2, b2)
    gather_k, nw, chunks = _build_gather()
    idx = text.T.reshape(nw, chunks, _GLANES)
    out = gather_k(idx, y)
    return out.reshape(_SEQ, _BATCH).T.reshape(_BATCH, _SEQ, 1)


# speedup vs baseline: 130.9645x; 1.0412x over previous
import functools

import jax
import jax.numpy as jnp
from jax import lax
from jax.experimental import pallas as pl
from jax.experimental.pallas import tpu as pltpu
from jax.experimental.pallas import tpu_sc as plsc

_VOCAB = 1000000
_EMBED = 32
_HIDDEN = 128
_BATCH = 4096
_SEQ = 200
_NTOK = _BATCH * _SEQ

_BLK = 16384
_NBLK = (_VOCAB + _BLK - 1) // _BLK

_GLANES = 128


def _mlp_body(tt_ref, w1t_ref, b1_ref, w2c_ref, b2_ref, y_ref):
    tt = tt_ref[...]
    h = jnp.dot(w1t_ref[...], tt, preferred_element_type=jnp.float32)
    h = jnp.maximum(h + b1_ref[...], 0.0)
    y = jnp.sum(h * w2c_ref[...], axis=0) + b2_ref[0, 0]
    y_ref[...] = y


def _precompute_y(table, W1, b1, W2, b2):
    y = pl.pallas_call(
        _mlp_body,
        grid=(_NBLK,),
        in_specs=[
            pl.BlockSpec((_EMBED, _BLK), lambda i: (0, i)),
            pl.BlockSpec((_HIDDEN, _EMBED), lambda i: (0, 0)),
            pl.BlockSpec((_HIDDEN, 1), lambda i: (0, 0)),
            pl.BlockSpec((_HIDDEN, 1), lambda i: (0, 0)),
            pl.BlockSpec((1, 1), lambda i: (0, 0)),
        ],
        out_specs=pl.BlockSpec((_BLK,), lambda i: (i,)),
        out_shape=jax.ShapeDtypeStruct((_VOCAB,), jnp.float32),
    )(table.T, W1.T, b1.reshape(_HIDDEN, 1), W2, b2.reshape(1, 1))
    return y


@functools.lru_cache(maxsize=None)
def _build_gather():
    info = plsc.get_sparse_core_info()
    nc, ns = info.num_cores, info.num_subcores
    nw = nc * ns
    cols = _BATCH // nw
    mesh = plsc.VectorSubcoreMesh(core_axis_name="c", subcore_axis_name="s")

    @functools.partial(
        pl.kernel,
        mesh=mesh,
        out_type=jax.ShapeDtypeStruct((_SEQ, _BATCH), jnp.float32),
        scratch_types=[
            pltpu.VMEM((_SEQ, cols), jnp.int32),
            pltpu.VMEM((_SEQ, cols), jnp.float32),
            pltpu.SemaphoreType.DMA,
        ],
    )
    def gather_k(idx_hbm, tab_hbm, out_hbm, idx_v, vals_v, sem):
        wid = lax.axis_index("s") * nc + lax.axis_index("c")
        base = wid * cols
        pltpu.sync_copy(idx_hbm.at[:, pl.ds(base, cols)], idx_v)

        def fire(j, c):
            pltpu.make_async_copy(
                tab_hbm.at[idx_v.at[j]], vals_v.at[j], sem).start()
            return c

        lax.fori_loop(0, _SEQ, fire, 0)

        def drain(j, c):
            pltpu.make_async_copy(
                tab_hbm.at[idx_v.at[j]], vals_v.at[j], sem).wait()
            return c

        lax.fori_loop(0, _SEQ, drain, 0)
        pltpu.sync_copy(vals_v, out_hbm.at[:, pl.ds(base, cols)])

    return gather_k


def kernel(text, embedding_table, W1, b1, W2, b2):
    y = _precompute_y(embedding_table, W1, b1, W2, b2)
    gather_k = _build_gather()
    out = gather_k(text.T, y)
    return out.T.reshape(_BATCH, _SEQ, 1)


# speedup vs baseline: 139.5867x; 1.0658x over previous
import functools

import jax
import jax.numpy as jnp
from jax import lax
from jax.experimental import pallas as pl
from jax.experimental.pallas import tpu as pltpu
from jax.experimental.pallas import tpu_sc as plsc

_VOCAB = 1000000
_EMBED = 32
_HIDDEN = 128
_BATCH = 4096
_SEQ = 200
_NTOK = _BATCH * _SEQ

_BLK = 32768
_NBLK = (_VOCAB + _BLK - 1) // _BLK

_GLANES = 128


def _mlp_body(tt_ref, w1t_ref, b1_ref, w2c_ref, b2_ref, y_ref):
    tt = tt_ref[...]
    h = jnp.dot(w1t_ref[...], tt, preferred_element_type=jnp.float32)
    h = jnp.maximum(h + b1_ref[...], 0.0)
    y = jnp.sum(h * w2c_ref[...], axis=0) + b2_ref[0, 0]
    y_ref[...] = y


def _precompute_y(table, W1, b1, W2, b2):
    y = pl.pallas_call(
        _mlp_body,
        grid=(_NBLK,),
        in_specs=[
            pl.BlockSpec((_EMBED, _BLK), lambda i: (0, i)),
            pl.BlockSpec((_HIDDEN, _EMBED), lambda i: (0, 0)),
            pl.BlockSpec((_HIDDEN, 1), lambda i: (0, 0)),
            pl.BlockSpec((_HIDDEN, 1), lambda i: (0, 0)),
            pl.BlockSpec((1, 1), lambda i: (0, 0)),
        ],
        out_specs=pl.BlockSpec((_BLK,), lambda i: (i,)),
        out_shape=jax.ShapeDtypeStruct((_VOCAB,), jnp.float32),
    )(table.T, W1.T, b1.reshape(_HIDDEN, 1), W2, b2.reshape(1, 1))
    return y


@functools.lru_cache(maxsize=None)
def _build_gather():
    info = plsc.get_sparse_core_info()
    nc, ns = info.num_cores, info.num_subcores
    nw = nc * ns
    cols = _BATCH // nw
    mesh = plsc.VectorSubcoreMesh(core_axis_name="c", subcore_axis_name="s")

    @functools.partial(
        pl.kernel,
        mesh=mesh,
        out_type=jax.ShapeDtypeStruct((_SEQ, _BATCH), jnp.float32),
        scratch_types=[
            pltpu.VMEM((_SEQ, cols), jnp.int32),
            pltpu.VMEM((_SEQ, cols), jnp.float32),
            pltpu.SemaphoreType.DMA,
        ],
    )
    def gather_k(idx_hbm, tab_hbm, out_hbm, idx_v, vals_v, sem):
        wid = lax.axis_index("s") * nc + lax.axis_index("c")
        base = wid * cols
        pltpu.sync_copy(idx_hbm.at[:, pl.ds(base, cols)], idx_v)

        def fire(j, c):
            pltpu.make_async_copy(
                tab_hbm.at[idx_v.at[j]], vals_v.at[j], sem).start()
            return c

        lax.fori_loop(0, _SEQ, fire, 0)

        def drain(j, c):
            pltpu.make_async_copy(
                tab_hbm.at[idx_v.at[j]], vals_v.at[j], sem).wait()
            return c

        lax.fori_loop(0, _SEQ, drain, 0)
        pltpu.sync_copy(vals_v, out_hbm.at[:, pl.ds(base, cols)])

    return gather_k


def kernel(text, embedding_table, W1, b1, W2, b2):
    y = _precompute_y(embedding_table, W1, b1, W2, b2)
    gather_k = _build_gather()
    out = gather_k(text.T, y)
    return out.T.reshape(_BATCH, _SEQ, 1)
